# Initial kernel scaffold; baseline (speedup 1.0000x reference)
import functools

import jax
import jax.numpy as jnp
from jax import lax
from jax.experimental import pallas as pl
from jax.experimental.pallas import tpu as pltpu
from jax.experimental.pallas import tpu_sc as plsc

_D = 32
_NW = 32
_CHUNK = 2048


@functools.cache
def _make_gather(B: int):
    b_per_w = B // _NW
    n_chunk = b_per_w // _CHUNK
    mesh = plsc.VectorSubcoreMesh(core_axis_name="c", subcore_axis_name="s")

    @functools.partial(
        pl.kernel,
        out_type=jax.ShapeDtypeStruct((B, _D), jnp.float32),
        mesh=mesh,
        scratch_types=[
            pltpu.VMEM((_CHUNK,), jnp.int32),
            pltpu.VMEM((_CHUNK, _D), jnp.float32),
            pltpu.SemaphoreType.DMA,
        ],
    )
    def gather_kernel(idx_hbm, table_hbm, out_hbm, idx_v, rows_v, sem):
        wid = lax.axis_index("s") * 2 + lax.axis_index("c")
        base = wid * b_per_w

        def body(i, carry):
            off = base + i * _CHUNK
            pltpu.sync_copy(idx_hbm.at[pl.ds(off, _CHUNK)], idx_v)
            pltpu.async_copy(table_hbm.at[idx_v], rows_v, sem).wait()
            pltpu.sync_copy(rows_v, out_hbm.at[pl.ds(off, _CHUNK)])
            return carry

        lax.fori_loop(0, n_chunk, body, 0)

    return gather_kernel


def kernel(genre_labels, table):
    b, h = genre_labels.shape
    idx = genre_labels.reshape(-1).astype(jnp.int32)
    out = _make_gather(b * h)(idx, table)
    return out.reshape(b, h, _D)


# speedup vs baseline: 1.0946x; 1.0946x over previous
import functools

import jax
import jax.numpy as jnp
from jax import lax
from jax.experimental import pallas as pl
from jax.experimental.pallas import tpu as pltpu
from jax.experimental.pallas import tpu_sc as plsc

_D = 32
_NW = 32
_G = 128
_NG = 8
_CHUNK = _G * _NG


@functools.cache
def _make_gather(B: int):
    b_per_w = B // _NW
    n_chunk = b_per_w // _CHUNK
    assert B % _NW == 0 and b_per_w % _CHUNK == 0, (B, b_per_w)
    mesh = plsc.VectorSubcoreMesh(core_axis_name="c", subcore_axis_name="s")

    @functools.partial(
        pl.kernel,
        out_type=jax.ShapeDtypeStruct((B, _D), jnp.float32),
        mesh=mesh,
        scratch_types=[
            pltpu.VMEM((_NG, _G), jnp.int32),
            pltpu.VMEM((_CHUNK, _D), jnp.float32),
            pltpu.SemaphoreType.DMA,
        ],
        compiler_params=pltpu.CompilerParams(use_tc_tiling_on_sc=False),
    )
    def gather_kernel(idx_hbm, table_hbm, out_hbm, idx_v, rows_v, sem):
        wid = lax.axis_index("s") * 2 + lax.axis_index("c")
        base = wid * b_per_w

        def body(i, carry):
            off = base + i * _CHUNK
            pltpu.sync_copy(idx_hbm.at[pl.ds(off // _G, _NG)], idx_v)
            descs = [
                pltpu.async_copy(
                    table_hbm.at[idx_v.at[j]],
                    rows_v.at[pl.ds(j * _G, _G)],
                    sem,
                )
                for j in range(_NG)
            ]
            for d in descs:
                d.wait()
            pltpu.sync_copy(rows_v, out_hbm.at[pl.ds(off, _CHUNK)])
            return carry

        lax.fori_loop(0, n_chunk, body, 0)

    return gather_kernel


def kernel(genre_labels, table):
    b, h = genre_labels.shape
    idx = genre_labels.reshape(-1, _G).astype(jnp.int32)
    out = _make_gather(b * h)(idx, table)
    return out.reshape(b, h, _D)


# speedup vs baseline: 1.2951x; 1.1831x over previous
import functools

import jax
import jax.numpy as jnp
from jax import lax
from jax.experimental import pallas as pl
from jax.experimental.pallas import tpu as pltpu
from jax.experimental.pallas import tpu_sc as plsc

_D = 32
_NW = 32
_NB = 256
_NGI = _NB // 128


@functools.cache
def _make_kernel(BATCH: int, HIST: int):
    nb_strip = BATCH // _NW
    n_half = nb_strip // _NB
    mesh = plsc.VectorSubcoreMesh(core_axis_name="c", subcore_axis_name="s")

    @functools.partial(
        pl.kernel,
        out_type=jax.ShapeDtypeStruct((HIST, _D, BATCH), jnp.float32),
        mesh=mesh,
        scratch_types=[
            pltpu.VMEM((_NB,), jnp.int32),
            pltpu.VMEM((_NGI, 128), jnp.int32),
            pltpu.VMEM((_NB,), jnp.int32),
            pltpu.VMEM((_NB, 128), jnp.float32),
            pltpu.VMEM((_D, _NB), jnp.float32),
            pltpu.SemaphoreType.DMA,
        ],
        compiler_params=pltpu.CompilerParams(needs_layout_passes=False),
    )
    def gather_kernel(idx_hbm, table2_hbm, out_hbm,
                      idxb, gi, qoff, rows_v, trans_v, sem):
        wid = lax.axis_index("s") * 2 + lax.axis_index("c")
        b0 = wid * nb_strip
        iota16 = lax.iota(jnp.int32, 16)

        def chunk_body(c, carry):
            h = c // n_half
            half = c - h * n_half
            pos = h * BATCH + b0 + half * _NB

            pltpu.sync_copy(idx_hbm.at[pl.ds(pos, _NB)], idxb)

            def prep(t, carry2):
                v = idxb[pl.ds(t * 16, 16)]
                gi[t // 8, pl.ds((t % 8) * 16, 16)] = lax.shift_right_logical(v, 2)
                qoff[pl.ds(t * 16, 16)] = lax.shift_left(jnp.bitwise_and(v, 3), 5)
                return carry2

            lax.fori_loop(0, _NB // 16, prep, 0)

            descs = [
                pltpu.async_copy(
                    table2_hbm.at[gi.at[j]],
                    rows_v.at[pl.ds(j * 128, 128)],
                    sem,
                )
                for j in range(_NGI)
            ]
            for d in descs:
                d.wait()

            def transpose_block(bb, carry2):
                row_ids = bb * 16 + iota16
                col_base = qoff[pl.ds(bb * 16, 16)]

                def col_loop(dd, carry3):
                    vals = plsc.load_gather(rows_v, [row_ids, col_base + dd])
                    trans_v[dd, pl.ds(bb * 16, 16)] = vals
                    return carry3

                lax.fori_loop(0, _D, col_loop, 0)
                return carry2

            lax.fori_loop(0, _NB // 16, transpose_block, 0)

            pltpu.sync_copy(trans_v, out_hbm.at[h, :, pl.ds(b0 + half * _NB, _NB)])
            return carry

        lax.fori_loop(0, HIST * n_half, chunk_body, 0)

    return gather_kernel


def kernel(genre_labels, table):
    b, h = genre_labels.shape
    idx = jnp.ravel(genre_labels.T).astype(jnp.int32)
    table2 = table.reshape(table.shape[0] // 4, 4 * table.shape[1])
    out = _make_kernel(b, h)(idx, table2)
    return out.transpose(2, 0, 1)


# speedup vs baseline: 1.5028x; 1.1604x over previous
import functools

import jax
import jax.numpy as jnp
from jax import lax
from jax.experimental import pallas as pl
from jax.experimental.pallas import tpu as pltpu
from jax.experimental.pallas import tpu_sc as plsc

_D = 32
_NW = 32
_NB = 256
_NGI = _NB // 128


@functools.cache
def _make_kernel(BATCH: int, HIST: int):
    nb_strip = BATCH // _NW
    n_half = nb_strip // _NB
    n_chunk = HIST * n_half
    mesh = plsc.VectorSubcoreMesh(core_axis_name="c", subcore_axis_name="s")

    @functools.partial(
        pl.kernel,
        out_type=jax.ShapeDtypeStruct((HIST, _D, BATCH), jnp.float32),
        mesh=mesh,
        scratch_types=[
            pltpu.VMEM((2, _NB), jnp.int32),
            pltpu.VMEM((2 * _NGI, 128), jnp.int32),
            pltpu.VMEM((2, _NB), jnp.int32),
            pltpu.VMEM((2, _NB, 128), jnp.float32),
            pltpu.VMEM((_D, _NB), jnp.float32),
            pltpu.SemaphoreType.DMA,
            pltpu.SemaphoreType.DMA,
        ],
        compiler_params=pltpu.CompilerParams(needs_layout_passes=False),
    )
    def gather_kernel(idx_hbm, table2_hbm, out_hbm,
                      idxb, gi, qoff, rows_v, trans_v, sem0, sem1):
        wid = lax.axis_index("s") * 2 + lax.axis_index("c")
        b0 = wid * nb_strip
        iota16 = lax.iota(jnp.int32, 16)
        sems = (sem0, sem1)

        def fetch(c, buf):
            h = c // n_half
            pos = h * BATCH + b0 + (c - h * n_half) * _NB
            pltpu.sync_copy(idx_hbm.at[pl.ds(pos, _NB)], idxb.at[buf])
            for t in range(_NB // 16):
                v = idxb[buf, pl.ds(t * 16, 16)]
                gi[buf * _NGI + t // 8, pl.ds((t % 8) * 16, 16)] = (
                    lax.shift_right_logical(v, 2))
                qoff[buf, pl.ds(t * 16, 16)] = (
                    lax.shift_left(jnp.bitwise_and(v, 3), 5))
            for j in range(_NGI):
                pltpu.async_copy(
                    table2_hbm.at[gi.at[buf * _NGI + j]],
                    rows_v.at[buf, pl.ds(j * 128, 128)],
                    sems[buf],
                )

        def drain(buf):
            for j in range(_NGI):
                pltpu.make_async_copy(
                    table2_hbm.at[gi.at[buf * _NGI + j]],
                    rows_v.at[buf, pl.ds(j * 128, 128)],
                    sems[buf],
                ).wait()

        def emit(c, buf):
            for bb in range(_NB // 16):
                row_ids = bb * 16 + iota16
                col_base = qoff[buf, pl.ds(bb * 16, 16)]
                for dd in range(_D):
                    vals = plsc.load_gather(
                        rows_v.at[buf], [row_ids, col_base + dd])
                    trans_v[dd, pl.ds(bb * 16, 16)] = vals
            h = c // n_half
            bpos = b0 + (c - h * n_half) * _NB
            pltpu.sync_copy(trans_v, out_hbm.at[h, :, pl.ds(bpos, _NB)])

        fetch(0, 0)

        def body(g, carry):
            c = 2 * g

            @pl.when(c + 1 < n_chunk)
            def _():
                fetch(c + 1, 1)

            drain(0)
            emit(c, 0)

            @pl.when(c + 2 < n_chunk)
            def _():
                fetch(c + 2, 0)

            @pl.when(c + 1 < n_chunk)
            def _():
                drain(1)
                emit(c + 1, 1)

            return carry

        lax.fori_loop(0, (n_chunk + 1) // 2, body, 0)

    return gather_kernel


def kernel(genre_labels, table):
    b, h = genre_labels.shape
    idx = jnp.ravel(genre_labels.T).astype(jnp.int32)
    table2 = table.reshape(table.shape[0] // 4, 4 * table.shape[1])
    out = _make_kernel(b, h)(idx, table2)
    return out.transpose(2, 0, 1)


# speedup vs baseline: 2.4067x; 1.6015x over previous
import functools

import jax
import jax.numpy as jnp
from jax import lax
from jax.experimental import pallas as pl
from jax.experimental.pallas import tpu as pltpu
from jax.experimental.pallas import tpu_sc as plsc

_D = 32
_NW = 32
_NB = 256
_NGI = _NB // 128
_TP = _NB + 2


@functools.cache
def _make_kernel(BATCH: int, HIST: int):
    nb_strip = BATCH // _NW
    n_half = nb_strip // _NB
    n_chunk = HIST * n_half
    mesh = plsc.VectorSubcoreMesh(core_axis_name="c", subcore_axis_name="s")

    @functools.partial(
        pl.kernel,
        out_type=jax.ShapeDtypeStruct((HIST, _D, BATCH), jnp.float32),
        mesh=mesh,
        scratch_types=[
            pltpu.VMEM((HIST, nb_strip), jnp.int32),
            pltpu.VMEM((2 * _NGI, 128), jnp.int32),
            pltpu.VMEM((2, _NB), jnp.int32),
            pltpu.VMEM((2, _NB, 128), jnp.float32),
            pltpu.VMEM((_D, _TP), jnp.float32),
            pltpu.SemaphoreType.DMA,
            pltpu.SemaphoreType.DMA,
        ],
        compiler_params=pltpu.CompilerParams(needs_layout_passes=False),
    )
    def gather_kernel(idx_hbm, table2_hbm, out_hbm,
                      idx_local, gi, qoff, rows_v, trans_v, sem0, sem1):
        wid = lax.axis_index("s") * 2 + lax.axis_index("c")
        b0 = wid * nb_strip
        iota16 = lax.iota(jnp.int32, 16)
        sems = (sem0, sem1)

        pltpu.sync_copy(idx_hbm.at[:, pl.ds(b0, nb_strip)], idx_local)

        def fetch(c, buf):
            h = c // n_half
            off = (c - h * n_half) * _NB
            for t in range(_NB // 16):
                v = idx_local[h, pl.ds(off + t * 16, 16)]
                gi[buf * _NGI + t // 8, pl.ds((t % 8) * 16, 16)] = (
                    lax.shift_right_logical(v, 2))
                qoff[buf, pl.ds(t * 16, 16)] = (
                    lax.shift_left(jnp.bitwise_and(v, 3), 5))
            for j in range(_NGI):
                pltpu.async_copy(
                    table2_hbm.at[gi.at[buf * _NGI + j]],
                    rows_v.at[buf, pl.ds(j * 128, 128)],
                    sems[buf],
                )

        def drain(buf):
            for j in range(_NGI):
                pltpu.make_async_copy(
                    table2_hbm.at[gi.at[buf * _NGI + j]],
                    rows_v.at[buf, pl.ds(j * 128, 128)],
                    sems[buf],
                ).wait()

        skews = [jnp.bitwise_and(dd + iota16, 15) for dd in range(16)]

        def emit(c, buf):
            def block(bg, carry):
                base = bg * 16
                row_ids = base + iota16
                qv = qoff[buf, pl.ds(base, 16)]
                for d0 in range(0, _D, 16):
                    for dd in range(16):
                        vals = plsc.load_gather(
                            rows_v.at[buf], [row_ids, qv + (d0 + skews[dd])])
                        plsc.store_scatter(
                            trans_v, [d0 + skews[dd], row_ids], vals)
                return carry

            lax.fori_loop(0, _NB // 16, block, 0)
            h = c // n_half
            bpos = b0 + (c - h * n_half) * _NB
            pltpu.sync_copy(trans_v.at[:, pl.ds(0, _NB)],
                            out_hbm.at[h, :, pl.ds(bpos, _NB)])

        fetch(0, 0)

        def body(g, carry):
            c = 2 * g

            @pl.when(c + 1 < n_chunk)
            def _():
                fetch(c + 1, 1)

            drain(0)
            emit(c, 0)

            @pl.when(c + 2 < n_chunk)
            def _():
                fetch(c + 2, 0)

            @pl.when(c + 1 < n_chunk)
            def _():
                drain(1)
                emit(c + 1, 1)

            return carry

        lax.fori_loop(0, (n_chunk + 1) // 2, body, 0)

    return gather_kernel


def kernel(genre_labels, table):
    b, h = genre_labels.shape
    idx2 = genre_labels.T.astype(jnp.int32)
    table2 = table.reshape(table.shape[0] // 4, 4 * table.shape[1])
    out = _make_kernel(b, h)(idx2, table2)
    return out.transpose(2, 0, 1)
